# baseline (speedup 1.0000x reference)
import functools

import jax
import jax.numpy as jnp
from jax import lax
from jax.experimental import pallas as pl
from jax.experimental.pallas import tpu as pltpu
from jax.experimental.pallas import tpu_sc as plsc

VOCAB = 32
B = 4096
L = 200
NW = 32
LB = B // NW
LC = 10
N_LCH = L // LC
NG = LB // 16


def _table_body(e_ref, wt_ref, b_ref, t_ref):
    t_ref[...] = (
        jnp.dot(e_ref[...], wt_ref[...], preferred_element_type=jnp.float32)
        + b_ref[...]
    )


def _make_table(embed_table, wt, b2):
    return pl.pallas_call(
        _table_body,
        out_shape=jax.ShapeDtypeStruct((VOCAB, VOCAB), jnp.float32),
    )(embed_table, wt, b2)


def _gather_body(t_hbm, ids_hbm, out_hbm, t_v, idx_v, sidx, buf0, buf1, s0, s1):
    wid = lax.axis_index("s") * 2 + lax.axis_index("c")
    b0 = wid * LB

    pltpu.sync_copy(t_hbm, t_v)
    pltpu.sync_copy(ids_hbm.at[pl.ds(b0 * L, LB * L)], idx_v)

    lane_b = lax.iota(jnp.int32, 16) * L

    def fill(buf, k):
        l0 = k * LC

        @plsc.parallel_loop(0, LC * NG, unroll=2)
        def _(q):
            i = q // NG
            g = q - i * NG
            ids16 = plsc.load_gather(idx_v, [lane_b + (g * 16 * L + l0 + i)])
            sidx[pl.ds(q * 16, 16)] = ids16 * VOCAB

        @plsc.parallel_loop(0, LC * NG * VOCAB, unroll=4)
        def _(q2):
            q = q2 // VOCAB
            v = q2 - q * VOCAB
            i = q // NG
            g = q - i * NG
            flat = sidx[pl.ds(q * 16, 16)] + v
            buf[i, v, pl.ds(g * 16, 16)] = plsc.load_gather(t_v, [flat])

    def store(buf, k, sem):
        return pltpu.make_async_copy(
            buf, out_hbm.at[pl.ds(k * LC, LC), :, pl.ds(b0, LB)], sem
        )

    fill(buf0, 0)
    store(buf0, 0, s0).start()

    def pair(p, carry):
        k1 = 2 * p + 1
        k2 = 2 * p + 2
        fill(buf1, k1)
        store(buf0, 2 * p, s0).wait()
        store(buf1, k1, s1).start()
        fill(buf0, k2)
        store(buf1, k1, s1).wait()
        store(buf0, k2, s0).start()
        return carry

    lax.fori_loop(0, (N_LCH - 2) // 2, pair, 0)

    fill(buf1, N_LCH - 1)
    store(buf0, N_LCH - 2, s0).wait()
    store(buf1, N_LCH - 1, s1).start()
    store(buf1, N_LCH - 1, s1).wait()


_gather = functools.partial(
    pl.kernel,
    out_type=jax.ShapeDtypeStruct((L, VOCAB, B), jnp.float32),
    mesh=plsc.VectorSubcoreMesh(core_axis_name="c", subcore_axis_name="s"),
    scratch_types=[
        pltpu.VMEM((VOCAB * VOCAB,), jnp.float32),
        pltpu.VMEM((LB * L,), jnp.int32),
        pltpu.VMEM((LC * NG * 16,), jnp.int32),
        pltpu.VMEM((LC, VOCAB, LB), jnp.float32),
        pltpu.VMEM((LC, VOCAB, LB), jnp.float32),
        pltpu.SemaphoreType.DMA,
        pltpu.SemaphoreType.DMA,
    ],
    compiler_params=pltpu.CompilerParams(
        use_tc_tiling_on_sc=True, needs_layout_passes=False
    ),
)(_gather_body)


def kernel(input_ids, embed_table, W, b):
    table = _make_table(embed_table, W.T, b.reshape(1, VOCAB))
    ids = input_ids.reshape(B * L).astype(jnp.int32)
    xt = _gather(table.reshape(VOCAB * VOCAB), ids)
    return jnp.transpose(xt, (2, 0, 1))


# speedup vs baseline: 1.2813x; 1.2813x over previous
import functools

import jax
import jax.numpy as jnp
from jax import lax
from jax.experimental import pallas as pl
from jax.experimental.pallas import tpu as pltpu
from jax.experimental.pallas import tpu_sc as plsc

VOCAB = 32
B = 4096
L = 200
NW = 32
LB = B // NW
LC = 10
N_LCH = L // LC
NG = LB // 16


def _table_body(e_ref, wt_ref, b_ref, t_ref):
    t_ref[...] = (
        jnp.dot(e_ref[...], wt_ref[...], preferred_element_type=jnp.float32)
        + b_ref[...]
    )


def _make_table(embed_table, wt, b2):
    return pl.pallas_call(
        _table_body,
        out_shape=jax.ShapeDtypeStruct((VOCAB, VOCAB), jnp.float32),
    )(embed_table, wt, b2)


def _gather_body(t_hbm, ids_hbm, out_hbm, t_v, idx_v, sidx, buf0, buf1, s0, s1):
    wid = lax.axis_index("s") * 2 + lax.axis_index("c")
    b0 = wid * LB

    pltpu.sync_copy(t_hbm, t_v)
    pltpu.sync_copy(ids_hbm.at[pl.ds(b0 * L, LB * L)], idx_v)

    lane_b = lax.iota(jnp.int32, 16) * L

    def fill(buf, k):
        l0 = k * LC

        @plsc.parallel_loop(0, LC * NG, unroll=2)
        def _(q):
            i = q // NG
            g = q - i * NG
            ids16 = plsc.load_gather(idx_v, [lane_b + (g * 16 * L + l0 + i)])
            flat = ids16 * VOCAB
            vals = [plsc.load_gather(t_v, [flat + v]) for v in range(VOCAB)]
            for v in range(VOCAB):
                buf[i, v, pl.ds(g * 16, 16)] = vals[v]

    def store(buf, k, sem):
        return pltpu.make_async_copy(
            buf, out_hbm.at[pl.ds(k * LC, LC), :, pl.ds(b0, LB)], sem
        )

    fill(buf0, 0)
    store(buf0, 0, s0).start()

    def pair(p, carry):
        k1 = 2 * p + 1
        k2 = 2 * p + 2
        fill(buf1, k1)
        store(buf0, 2 * p, s0).wait()
        store(buf1, k1, s1).start()
        fill(buf0, k2)
        store(buf1, k1, s1).wait()
        store(buf0, k2, s0).start()
        return carry

    lax.fori_loop(0, (N_LCH - 2) // 2, pair, 0)

    fill(buf1, N_LCH - 1)
    store(buf0, N_LCH - 2, s0).wait()
    store(buf1, N_LCH - 1, s1).start()
    store(buf1, N_LCH - 1, s1).wait()


_gather = functools.partial(
    pl.kernel,
    out_type=jax.ShapeDtypeStruct((L, VOCAB, B), jnp.float32),
    mesh=plsc.VectorSubcoreMesh(core_axis_name="c", subcore_axis_name="s"),
    scratch_types=[
        pltpu.VMEM((VOCAB * VOCAB,), jnp.float32),
        pltpu.VMEM((LB * L,), jnp.int32),
        pltpu.VMEM((LC * NG * 16,), jnp.int32),
        pltpu.VMEM((LC, VOCAB, LB), jnp.float32),
        pltpu.VMEM((LC, VOCAB, LB), jnp.float32),
        pltpu.SemaphoreType.DMA,
        pltpu.SemaphoreType.DMA,
    ],
    compiler_params=pltpu.CompilerParams(
        use_tc_tiling_on_sc=True, needs_layout_passes=False
    ),
)(_gather_body)


def kernel(input_ids, embed_table, W, b):
    table = _make_table(embed_table, W.T, b.reshape(1, VOCAB))
    ids = input_ids.reshape(B * L).astype(jnp.int32)
    xt = _gather(table.reshape(VOCAB * VOCAB), ids)
    return jnp.transpose(xt, (2, 0, 1))


# speedup vs baseline: 3.4066x; 2.6588x over previous
import functools

import jax
import jax.numpy as jnp
from jax import lax
from jax.experimental import pallas as pl
from jax.experimental.pallas import tpu as pltpu
from jax.experimental.pallas import tpu_sc as plsc

VOCAB = 32
TS = 33
B = 4096
L = 200
NW = 32
LB = B // NW
LC = 10
N_LCH = L // LC
NG = LB // 16


def _table_body(e_ref, wt_ref, b_ref, t_ref):
    t_ref[...] = (
        jnp.dot(e_ref[...], wt_ref[...], preferred_element_type=jnp.float32)
        + b_ref[...]
    )


def _make_table(embed_table, wt, b2):
    return pl.pallas_call(
        _table_body,
        out_shape=jax.ShapeDtypeStruct((VOCAB, TS), jnp.float32),
    )(embed_table, wt, b2)


def _gather_body(t_hbm, ids_hbm, out_hbm, t_v, tmp, idst, buf0, buf1, s0, s1):
    wid = lax.axis_index("s") * 2 + lax.axis_index("c")
    b0 = wid * LB

    pltpu.sync_copy(t_hbm, t_v)

    lane = lax.iota(jnp.int32, 16)
    lane_l = lane * L

    def stage(bg, carry):
        pltpu.sync_copy(ids_hbm.at[pl.ds((b0 + bg * 16) * L, 16 * L)], tmp)

        @plsc.parallel_loop(0, L, unroll=4)
        def _(l):
            ids16 = plsc.load_gather(tmp, [lane_l + l])
            idst[pl.ds(l * LB + bg * 16, 16)] = ids16 * TS

        return carry

    lax.fori_loop(0, NG, stage, 0)

    def fill(buf, k):
        l0 = k * LC

        @plsc.parallel_loop(0, LC * NG, unroll=2)
        def _(q):
            i = q // NG
            g = q - i * NG
            flat = idst[pl.ds((l0 + i) * LB + g * 16, 16)]
            vals = [plsc.load_gather(t_v, [flat + v]) for v in range(VOCAB)]
            for v in range(VOCAB):
                buf[i, v, pl.ds(g * 16, 16)] = vals[v]

    def store(buf, k, sem):
        return pltpu.make_async_copy(
            buf, out_hbm.at[pl.ds(k * LC, LC), :, pl.ds(b0, LB)], sem
        )

    fill(buf0, 0)
    store(buf0, 0, s0).start()

    def pair(p, carry):
        k1 = 2 * p + 1
        k2 = 2 * p + 2
        fill(buf1, k1)
        store(buf0, 2 * p, s0).wait()
        store(buf1, k1, s1).start()
        fill(buf0, k2)
        store(buf1, k1, s1).wait()
        store(buf0, k2, s0).start()
        return carry

    lax.fori_loop(0, (N_LCH - 2) // 2, pair, 0)

    fill(buf1, N_LCH - 1)
    store(buf0, N_LCH - 2, s0).wait()
    store(buf1, N_LCH - 1, s1).start()
    store(buf1, N_LCH - 1, s1).wait()


_gather = functools.partial(
    pl.kernel,
    out_type=jax.ShapeDtypeStruct((L, VOCAB, B), jnp.float32),
    mesh=plsc.VectorSubcoreMesh(core_axis_name="c", subcore_axis_name="s"),
    scratch_types=[
        pltpu.VMEM((VOCAB * TS,), jnp.float32),
        pltpu.VMEM((16 * L,), jnp.int32),
        pltpu.VMEM((LB * L,), jnp.int32),
        pltpu.VMEM((LC, VOCAB, LB), jnp.float32),
        pltpu.VMEM((LC, VOCAB, LB), jnp.float32),
        pltpu.SemaphoreType.DMA,
        pltpu.SemaphoreType.DMA,
    ],
    compiler_params=pltpu.CompilerParams(
        use_tc_tiling_on_sc=True, needs_layout_passes=False
    ),
)(_gather_body)


def kernel(input_ids, embed_table, W, b):
    wt33 = jnp.pad(W.T, ((0, 0), (0, TS - VOCAB)))
    b33 = jnp.pad(b.reshape(1, VOCAB), ((0, 0), (0, TS - VOCAB)))
    table = _make_table(embed_table, wt33, b33)
    ids = input_ids.reshape(B * L).astype(jnp.int32)
    xt = _gather(table.reshape(VOCAB * TS), ids)
    return jnp.transpose(xt, (2, 0, 1))
